# baseline (speedup 1.0000x reference)
import functools

import jax
import jax.numpy as jnp
from jax import lax
from jax.experimental import pallas as pl
from jax.experimental.pallas import tpu as pltpu
from jax.experimental.pallas import tpu_sc as plsc

ROWS = 4096
COLS = 8192
SC_ROWS = 928
TC_ROWS = ROWS - SC_ROWS
BR = 96
U32 = jnp.uint32

NC, NS = 2, 16
NW = NC * NS
SC_RPW = SC_ROWS // NW
VPR = COLS // 16


def _threefry_hash(x1):
    ks0 = U32(0)
    ks1 = U32(42)
    ks2 = U32(0x1BD11BDA ^ 42)
    rot0 = (13, 15, 26, 6)
    rot1 = (17, 29, 16, 24)

    def rounds(x0, x1, rots):
        for r in rots:
            x0 = x0 + x1
            x1 = (x1 << U32(r)) | (x1 >> U32(32 - r))
            x1 = x0 ^ x1
        return x0, x1

    x0 = jnp.zeros_like(x1)
    x1 = x1 + ks1
    x0, x1 = rounds(x0, x1, rot0)
    x0 = x0 + ks1
    x1 = x1 + (ks2 + U32(1))
    x0, x1 = rounds(x0, x1, rot1)
    x0 = x0 + ks2
    x1 = x1 + (ks0 + U32(2))
    x0, x1 = rounds(x0, x1, rot0)
    x0 = x0 + ks0
    x1 = x1 + (ks1 + U32(3))
    x0, x1 = rounds(x0, x1, rot1)
    x0 = x0 + ks1
    x1 = x1 + (ks2 + U32(4))
    x0, x1 = rounds(x0, x1, rot0)
    x0 = x0 + ks2
    x1 = x1 + (ks0 + U32(5))
    return x0 ^ x1


def _uniform_from_bits(bits):
    tiny = jnp.float32(jnp.finfo(jnp.float32).tiny)
    fb = (bits >> U32(9)) | U32(0x3F800000)
    floats = lax.bitcast_convert_type(fb, jnp.float32) - jnp.float32(1.0)
    return floats + tiny


def _tc_kernel(ratio_ref, out_ref):
    i = pl.program_id(0).astype(U32)
    base = i * U32(BR * COLS * 2)
    ii = lax.broadcasted_iota(U32, (BR, COLS), 0)
    jj = lax.broadcasted_iota(U32, (BR, COLS), 1)
    m0 = base + (ii * U32(COLS) + jj) * U32(2)
    l0 = jnp.log(_uniform_from_bits(_threefry_hash(m0)))
    l1 = jnp.log(_uniform_from_bits(_threefry_hash(m0 + U32(1))))
    ratio = ratio_ref[0]
    out_ref[...] = jnp.where(l0 < ratio * l1, 1, 0).astype(jnp.int32)


_L2C1 = jnp.float32(2.885390081777927)
_L2C3 = jnp.float32(2.885390081777927 / 3.0)
_L2C5 = jnp.float32(2.885390081777927 / 5.0)
_L2C7 = jnp.float32(2.885390081777927 / 7.0)
_SQRT2 = jnp.float32(1.4142135623730951)


def _log2_u(u):
    iu = lax.bitcast_convert_type(u, jnp.int32)
    e = (iu >> jnp.int32(23)) - jnp.int32(127)
    mbits = (iu & jnp.int32(0x7FFFFF)) | jnp.int32(0x3F800000)
    m = lax.bitcast_convert_type(mbits, jnp.float32)
    big = m > _SQRT2
    m = jnp.where(big, m * jnp.float32(0.5), m)
    e = jnp.where(big, e + jnp.int32(1), e)
    s = (m - jnp.float32(1.0)) / (m + jnp.float32(1.0))
    t = s * s
    p = _L2C5 + t * _L2C7
    p = _L2C3 + t * p
    p = _L2C1 + t * p
    return e.astype(jnp.float32) + s * p


def _sc_body(ratio_hbm, out_hbm, rvec, obuf):
    wid = lax.axis_index("s") * NC + lax.axis_index("c")
    pltpu.sync_copy(ratio_hbm, rvec)
    ratio = rvec[...]
    lane = lax.iota(U32, 16)

    def sample16(m0):
        g0 = _log2_u(_uniform_from_bits(_threefry_hash(m0)))
        g1 = _log2_u(_uniform_from_bits(_threefry_hash(m0 + U32(1))))
        return jnp.where(g0 < ratio * g1, jnp.int32(1), jnp.int32(0))

    def do_row(r, carry):
        row = wid * SC_RPW + r
        base = (U32(TC_ROWS) + row.astype(U32)) * U32(COLS * 2)

        @plsc.parallel_loop(0, VPR, unroll=2)
        def do_vec(v):
            mv = base + (v.astype(U32) * U32(16) + lane) * U32(2)
            obuf[pl.ds(v * 16, 16)] = sample16(mv)
        pltpu.sync_copy(obuf, out_hbm.at[row])
        return carry

    lax.fori_loop(0, SC_RPW, do_row, 0)


@functools.cache
def _sc_sample():
    return pl.kernel(
        _sc_body,
        out_type=jax.ShapeDtypeStruct((SC_ROWS, COLS), jnp.int32),
        mesh=plsc.VectorSubcoreMesh(core_axis_name="c", subcore_axis_name="s",
                                    num_cores=NC, num_subcores=NS),
        scratch_types=[
            pltpu.VMEM((16,), jnp.float32),
            pltpu.VMEM((COLS,), jnp.int32),
        ],
    )


@jax.jit
def _sample(logits):
    ratio = jnp.exp(logits[0] - logits[1]).astype(jnp.float32).reshape(1)
    tc_out = pl.pallas_call(
        _tc_kernel,
        grid=(TC_ROWS // BR,),
        in_specs=[pl.BlockSpec(memory_space=pltpu.SMEM)],
        out_specs=pl.BlockSpec((BR, COLS), lambda i: (i, 0)),
        out_shape=jax.ShapeDtypeStruct((ROWS, COLS), jnp.int32),
        compiler_params=pltpu.CompilerParams(
            dimension_semantics=("parallel",)),
    )(ratio)
    sc_out = _sc_sample()(jnp.broadcast_to(ratio, (16,)))
    return lax.dynamic_update_slice(tc_out, sc_out, (TC_ROWS, 0))


def kernel(logits_action, n):
    return _sample(logits_action)


# speedup vs baseline: 1.0032x; 1.0032x over previous
import functools

import jax
import jax.numpy as jnp
from jax import lax
from jax.experimental import pallas as pl
from jax.experimental.pallas import tpu as pltpu
from jax.experimental.pallas import tpu_sc as plsc

ROWS = 4096
COLS = 8192
SC_ROWS = 928
TC_ROWS = ROWS - SC_ROWS
BR = 96
U32 = jnp.uint32

NC, NS = 2, 16
NW = NC * NS
SC_RPW = SC_ROWS // NW
VPR = COLS // 16


def _threefry_hash(x1):
    ks0 = U32(0)
    ks1 = U32(42)
    ks2 = U32(0x1BD11BDA ^ 42)
    rot0 = (13, 15, 26, 6)
    rot1 = (17, 29, 16, 24)

    def rounds(x0, x1, rots):
        for r in rots:
            x0 = x0 + x1
            x1 = (x1 << U32(r)) | (x1 >> U32(32 - r))
            x1 = x0 ^ x1
        return x0, x1

    x0 = jnp.zeros_like(x1)
    x1 = x1 + ks1
    x0, x1 = rounds(x0, x1, rot0)
    x0 = x0 + ks1
    x1 = x1 + (ks2 + U32(1))
    x0, x1 = rounds(x0, x1, rot1)
    x0 = x0 + ks2
    x1 = x1 + (ks0 + U32(2))
    x0, x1 = rounds(x0, x1, rot0)
    x0 = x0 + ks0
    x1 = x1 + (ks1 + U32(3))
    x0, x1 = rounds(x0, x1, rot1)
    x0 = x0 + ks1
    x1 = x1 + (ks2 + U32(4))
    x0, x1 = rounds(x0, x1, rot0)
    x0 = x0 + ks2
    x1 = x1 + (ks0 + U32(5))
    return x0 ^ x1


def _uniform_from_bits(bits):
    tiny = jnp.float32(jnp.finfo(jnp.float32).tiny)
    fb = (bits >> U32(9)) | U32(0x3F800000)
    floats = lax.bitcast_convert_type(fb, jnp.float32) - jnp.float32(1.0)
    return floats + tiny


def _tc_kernel(ratio_ref, out_ref):
    i = pl.program_id(0).astype(U32)
    base = i * U32(BR * COLS * 2)
    ii = lax.broadcasted_iota(U32, (BR, COLS), 0)
    jj = lax.broadcasted_iota(U32, (BR, COLS), 1)
    m0 = base + (ii * U32(COLS) + jj) * U32(2)
    l0 = jnp.log(_uniform_from_bits(_threefry_hash(m0)))
    l1 = jnp.log(_uniform_from_bits(_threefry_hash(m0 + U32(1))))
    ratio = ratio_ref[0]
    out_ref[...] = jnp.where(l0 < ratio * l1, 1, 0).astype(jnp.int32)


_L2C1 = jnp.float32(2.885390081777927)
_L2C3 = jnp.float32(2.885390081777927 / 3.0)
_L2C5 = jnp.float32(2.885390081777927 / 5.0)
_L2C7 = jnp.float32(2.885390081777927 / 7.0)
_SQRT2 = jnp.float32(1.4142135623730951)


def _log2_u(u):
    iu = lax.bitcast_convert_type(u, jnp.int32)
    e = (iu >> jnp.int32(23)) - jnp.int32(127)
    mbits = (iu & jnp.int32(0x7FFFFF)) | jnp.int32(0x3F800000)
    m = lax.bitcast_convert_type(mbits, jnp.float32)
    big = m > _SQRT2
    m = jnp.where(big, m * jnp.float32(0.5), m)
    e = jnp.where(big, e + jnp.int32(1), e)
    s = (m - jnp.float32(1.0)) / (m + jnp.float32(1.0))
    t = s * s
    p = _L2C5 + t * _L2C7
    p = _L2C3 + t * p
    p = _L2C1 + t * p
    return e.astype(jnp.float32) + s * p


def _sc_body(ratio_hbm, out_hbm, rvec, obuf_a, obuf_b, sem_a, sem_b):
    wid = lax.axis_index("s") * NC + lax.axis_index("c")
    pltpu.sync_copy(ratio_hbm, rvec)
    ratio = rvec[...]
    lane = lax.iota(U32, 16)
    row0 = wid * SC_RPW

    def sample16(m0):
        g0 = _log2_u(_uniform_from_bits(_threefry_hash(m0)))
        g1 = _log2_u(_uniform_from_bits(_threefry_hash(m0 + U32(1))))
        return jnp.where(g0 < ratio * g1, jnp.int32(1), jnp.int32(0))

    def fill(buf, r):
        base = (U32(TC_ROWS) + r.astype(U32)) * U32(COLS * 2)

        @plsc.parallel_loop(0, VPR, unroll=2)
        def do_vec(v):
            mv = base + (v.astype(U32) * U32(16) + lane) * U32(2)
            buf[pl.ds(v * 16, 16)] = sample16(mv)

    fill(obuf_a, row0)
    pltpu.async_copy(obuf_a, out_hbm.at[row0], sem_a)

    def do_pair(p, carry):
        rb = row0 + 2 * p + 1
        ra = rb + 1
        fill(obuf_b, rb)
        pltpu.async_copy(obuf_b, out_hbm.at[rb], sem_b)
        pltpu.make_async_copy(obuf_a, out_hbm.at[row0], sem_a).wait()
        fill(obuf_a, ra)
        pltpu.async_copy(obuf_a, out_hbm.at[ra], sem_a)
        pltpu.make_async_copy(obuf_b, out_hbm.at[rb], sem_b).wait()
        return carry

    lax.fori_loop(0, (SC_RPW - 1) // 2, do_pair, 0)
    pltpu.make_async_copy(obuf_a, out_hbm.at[row0], sem_a).wait()


@functools.cache
def _sc_sample():
    return pl.kernel(
        _sc_body,
        out_type=jax.ShapeDtypeStruct((SC_ROWS, COLS), jnp.int32),
        mesh=plsc.VectorSubcoreMesh(core_axis_name="c", subcore_axis_name="s",
                                    num_cores=NC, num_subcores=NS),
        scratch_types=[
            pltpu.VMEM((16,), jnp.float32),
            pltpu.VMEM((COLS,), jnp.int32),
            pltpu.VMEM((COLS,), jnp.int32),
            pltpu.SemaphoreType.DMA,
            pltpu.SemaphoreType.DMA,
        ],
    )


@jax.jit
def _sample(logits):
    ratio = jnp.exp(logits[0] - logits[1]).astype(jnp.float32).reshape(1)
    tc_out = pl.pallas_call(
        _tc_kernel,
        grid=(TC_ROWS // BR,),
        in_specs=[pl.BlockSpec(memory_space=pltpu.SMEM)],
        out_specs=pl.BlockSpec((BR, COLS), lambda i: (i, 0)),
        out_shape=jax.ShapeDtypeStruct((ROWS, COLS), jnp.int32),
        compiler_params=pltpu.CompilerParams(
            dimension_semantics=("parallel",)),
    )(ratio)
    sc_out = _sc_sample()(jnp.broadcast_to(ratio, (16,)))
    return lax.dynamic_update_slice(tc_out, sc_out, (TC_ROWS, 0))


def kernel(logits_action, n):
    return _sample(logits_action)
